# baseline (speedup 1.0000x reference)
import functools

import jax
import jax.numpy as jnp
from jax import lax
from jax.experimental import pallas as pl
from jax.experimental.pallas import tpu as pltpu
from jax.experimental.pallas import tpu_sc as plsc

NPOL2 = 4
NBL = 128
NRED = 32
NROW = NPOL2 * NBL
NPROW = NPOL2 * NRED
D = 16 * 2048
C = 2048
NCHUNK = D // C
NWORK = 32
RPW = NROW // NWORK
L = 16


def _sc_body(v_hbm, p_hbm, vis_hbm, out_hbm, idx_v, vbuf, pbuf, obuf, sem):
    cid = lax.axis_index("c")
    sid = lax.axis_index("s")
    wid = sid * 2 + cid
    base = wid * RPW
    pq = base // NBL
    b0 = base % NBL

    pltpu.sync_copy(vis_hbm.at[pl.ds(b0, RPW)], idx_v)
    iv = idx_v[...]
    giv = (iv + pq * NRED) * NCHUNK

    def add_row(j, _):
        def add_vec(k, _):
            sl = pl.ds(k * L, L)
            obuf[j, sl] = vbuf[j, sl] + pbuf[j, sl]
            return 0
        lax.fori_loop(0, C // L, add_vec, 0)
        return 0

    for c in range(NCHUNK):
        cp = pltpu.async_copy(p_hbm.at[giv + c], pbuf, sem)
        pltpu.sync_copy(v_hbm.at[pl.ds(base, RPW), c], vbuf)
        cp.wait()
        for j in range(RPW):
            add_row(j, 0)
        pltpu.sync_copy(obuf, out_hbm.at[pl.ds(base, RPW), c])


_mesh = plsc.VectorSubcoreMesh(core_axis_name="c", subcore_axis_name="s")

_sc_kernel = functools.partial(
    pl.kernel,
    mesh=_mesh,
    out_type=jax.ShapeDtypeStruct((NROW, NCHUNK, C), jnp.float32),
    scratch_types=[
        pltpu.VMEM((RPW,), jnp.int32),
        pltpu.VMEM((RPW, C), jnp.float32),
        pltpu.VMEM((RPW, C), jnp.float32),
        pltpu.VMEM((RPW, C), jnp.float32),
        pltpu.SemaphoreType.DMA,
    ],
)(_sc_body)


@jax.jit
def kernel(V_m, params, vis2red):
    v = V_m.reshape(NROW, NCHUNK, C)
    p = params.reshape(NPROW * NCHUNK, C)
    out = _sc_kernel(v, p, vis2red.astype(jnp.int32))
    return out.reshape(V_m.shape)
